# Initial kernel scaffold; baseline (speedup 1.0000x reference)
import jax
import jax.numpy as jnp
from jax.experimental import pallas as pl


def kernel(features, edge_index, W1l, W1r, b1, W2l, W2r, b2, Wf1, bf1, Wf2, bf2):
    raise NotImplementedError("write your pallas kernel here")


# speedup vs baseline: 6.6581x; 6.6581x over previous
import jax
import jax.numpy as jnp
from jax import lax
from jax.experimental import pallas as pl
from jax.experimental.pallas import tpu as pltpu
from jax.experimental.pallas import tpu_sc as plsc

_N = 10000
_E = 160000
_D = 128
_H1 = 128
_H2 = 128
_F1 = 64
_F2 = 2
_EPS = 1e-07

_NC = 2
_NS = 16
_NW = _NC * _NS
_EPT = _E // _NW
_CH = 125
_NCHUNK = _EPT // _CH
_RPS = 624
_TBASE = _RPS * _NS
_TAIL = _N - _TBASE
_AIT = 313
_EPAD = _NW * _AIT * 16

_f32 = jnp.float32
_mesh = plsc.VectorSubcoreMesh(core_axis_name="c", subcore_axis_name="s")


def _sc_agg_body(x_hbm, src_hbm, dst_hbm, zf_hbm, out_feat,
                 sidx, didx, rows, feat_acc, sem):
    cid = lax.axis_index("c")
    sid = lax.axis_index("s")
    wid = sid * _NC + cid
    rb = sid * _RPS
    pltpu.sync_copy(zf_hbm.at[pl.ds(rb, _RPS)], feat_acc.at[pl.ds(rb, _RPS)])

    @pl.when(sid == _NS - 1)
    def _():
        pltpu.sync_copy(zf_hbm.at[pl.ds(_TBASE, _TAIL)],
                        feat_acc.at[pl.ds(_TBASE, _TAIL)])

    pltpu.sync_copy(src_hbm.at[wid], sidx)
    pltpu.sync_copy(dst_hbm.at[wid], didx)
    plsc.subcore_barrier()

    def chunk(j, carry):
        pltpu.async_copy(x_hbm.at[sidx.at[j]], rows, sem).wait()
        pltpu.sync_copy(rows, feat_acc.at[didx.at[j]], add=True)
        return carry

    lax.fori_loop(0, _NCHUNK, chunk, 0)
    plsc.subcore_barrier()
    pltpu.sync_copy(feat_acc.at[pl.ds(rb, _RPS)],
                    out_feat.at[cid, pl.ds(rb, _RPS)])

    @pl.when(sid == _NS - 1)
    def _():
        pltpu.sync_copy(feat_acc.at[pl.ds(_TBASE, _TAIL)],
                        out_feat.at[cid, pl.ds(_TBASE, _TAIL)])


_sc_agg = pl.kernel(
    _sc_agg_body,
    out_type=jax.ShapeDtypeStruct((_NC, _N, _D), _f32),
    mesh=_mesh,
    scratch_types=[
        pltpu.VMEM((_NCHUNK, _CH), jnp.int32),
        pltpu.VMEM((_NCHUNK, _CH), jnp.int32),
        pltpu.VMEM((_CH, _D), _f32),
        pltpu.VMEM_SHARED((_N, _D), _f32),
        pltpu.SemaphoreType.DMA,
    ],
)


def _sc_cnt_body(dst_hbm, zc_hbm, ones_hbm, out_cnt, didx, ones_b, cnt_acc):
    cid = lax.axis_index("c")
    sid = lax.axis_index("s")
    wid = sid * _NC + cid
    rb = sid * _RPS
    pltpu.sync_copy(zc_hbm.at[pl.ds(rb, _RPS)], cnt_acc.at[pl.ds(rb, _RPS)])

    @pl.when(sid == _NS - 1)
    def _():
        pltpu.sync_copy(zc_hbm.at[pl.ds(_TBASE, _TAIL)],
                        cnt_acc.at[pl.ds(_TBASE, _TAIL)])

    pltpu.sync_copy(ones_hbm, ones_b)
    pltpu.sync_copy(dst_hbm.at[wid], didx)
    plsc.subcore_barrier()

    def chunk(j, carry):
        pltpu.sync_copy(ones_b, cnt_acc.at[didx.at[j]], add=True)
        return carry

    lax.fori_loop(0, _NCHUNK, chunk, 0)
    plsc.subcore_barrier()
    pltpu.sync_copy(cnt_acc.at[pl.ds(rb, _RPS)],
                    out_cnt.at[cid, pl.ds(rb, _RPS)])

    @pl.when(sid == _NS - 1)
    def _():
        pltpu.sync_copy(cnt_acc.at[pl.ds(_TBASE, _TAIL)],
                        out_cnt.at[cid, pl.ds(_TBASE, _TAIL)])


_sc_cnt = pl.kernel(
    _sc_cnt_body,
    out_type=jax.ShapeDtypeStruct((_NC, _N, _D), _f32),
    mesh=_mesh,
    scratch_types=[
        pltpu.VMEM((_NCHUNK, _CH), jnp.int32),
        pltpu.VMEM((_CH, _D), _f32),
        pltpu.VMEM_SHARED((_N, _D), _f32),
    ],
)


def _sc_adj_body(a0_hbm, a1_hbm, src_hbm, dst_hbm, out_hbm,
                 a0b, a1b, sbuf, dbuf, outb):
    cid = lax.axis_index("c")
    sid = lax.axis_index("s")
    wid = sid * _NC + cid
    pltpu.sync_copy(a0_hbm, a0b)
    pltpu.sync_copy(a1_hbm, a1b)
    pltpu.sync_copy(src_hbm.at[wid], sbuf)
    pltpu.sync_copy(dst_hbm.at[wid], dbuf)
    lane = lax.iota(jnp.int32, 16)
    ebase = wid * (_AIT * 16)

    def it(i, carry):
        s00, s01, s10, s11 = carry
        idx_s = sbuf[i, :]
        idx_d = dbuf[i, :]
        a0s = plsc.load_gather(a0b, [idx_s])
        a1s = plsc.load_gather(a1b, [idx_s])
        a0d = plsc.load_gather(a0b, [idx_d])
        a1d = plsc.load_gather(a1b, [idx_d])
        w = jnp.where(ebase + i * 16 + lane < _E, 1.0, 0.0).astype(_f32)
        a0dw = a0d * w
        a1dw = a1d * w
        return (s00 + a0s * a0dw, s01 + a0s * a1dw,
                s10 + a1s * a0dw, s11 + a1s * a1dw)

    z = jnp.zeros((16,), _f32)
    s00, s01, s10, s11 = lax.fori_loop(0, _AIT, it, (z, z, z, z))
    outb[0, :] = s00
    outb[1, :] = s01
    outb[2, :] = s10
    outb[3, :] = s11
    pltpu.sync_copy(outb, out_hbm.at[wid])


_sc_adj = pl.kernel(
    _sc_adj_body,
    out_type=jax.ShapeDtypeStruct((_NW, 4, 16), _f32),
    mesh=_mesh,
    compiler_params=pltpu.CompilerParams(needs_layout_passes=False),
    scratch_types=[
        pltpu.VMEM((_N,), _f32),
        pltpu.VMEM((_N,), _f32),
        pltpu.VMEM((_AIT, 16), jnp.int32),
        pltpu.VMEM((_AIT, 16), jnp.int32),
        pltpu.VMEM((4, 16), _f32),
    ],
)

_BLK = 400
_GRID = _N // _BLK


def _tc1_body(x, agg, cnt, W1l, W1r, b1, out):
    c = jnp.maximum(cnt[0, :, 0:1] + cnt[1, :, 0:1], 1.0)
    mean = (agg[0] + agg[1]) / c
    h = (jnp.dot(mean, W1l[...], preferred_element_type=_f32)
         + jnp.dot(x[...], W1r[...], preferred_element_type=_f32) + b1[...])
    out[...] = jnp.maximum(h, 0.0)


def _tc1(x, agg, cnt, W1l, W1r, b1):
    return pl.pallas_call(
        _tc1_body,
        grid=(_GRID,),
        in_specs=[
            pl.BlockSpec((_BLK, _D), lambda i: (i, 0)),
            pl.BlockSpec((_NC, _BLK, _D), lambda i: (0, i, 0)),
            pl.BlockSpec((_NC, _BLK, _D), lambda i: (0, i, 0)),
            pl.BlockSpec((_D, _H1), lambda i: (0, 0)),
            pl.BlockSpec((_D, _H1), lambda i: (0, 0)),
            pl.BlockSpec((1, _H1), lambda i: (0, 0)),
        ],
        out_specs=pl.BlockSpec((_BLK, _H1), lambda i: (i, 0)),
        out_shape=jax.ShapeDtypeStruct((_N, _H1), _f32),
    )(x, agg, cnt, W1l, W1r, b1)


def _tc2_body(h1, agg, cnt, u, g, W2l, W2r, b2, Wf1, bf1, Wf2, bf2,
              assign, acc):
    i = pl.program_id(0)
    c = jnp.maximum(cnt[0, :, 0:1] + cnt[1, :, 0:1], 1.0)
    mean = (agg[0] + agg[1]) / c
    nf = (jnp.dot(mean, W2l[...], preferred_element_type=_f32)
          + jnp.dot(h1[...], W2r[...], preferred_element_type=_f32) + b2[...])
    ab = jnp.tanh(jnp.dot(nf, Wf1[...], preferred_element_type=_f32) + bf1[...])
    lg = jnp.dot(ab, Wf2[...], preferred_element_type=_f32) + bf2[...]
    l0 = lg[:, 0:1]
    l1 = lg[:, 1:2]
    m = jnp.maximum(l0, l1)
    e0 = jnp.exp(l0 - m)
    e1 = jnp.exp(l1 - m)
    s = e0 + e1
    a0 = e0 / s
    a1 = e1 / s
    assign[...] = jnp.concatenate([a0, a1], axis=1)
    q0 = a0 + g[:, 0:1]
    q1 = a1 + g[:, 1:2]
    mq = jnp.maximum(q0, q1)
    f0 = jnp.exp(q0 - mq)
    f1 = jnp.exp(q1 - mq)
    fs = f0 + f1
    lp = f0 / fs
    ln = f1 / fs

    def brd(v):
        return jnp.broadcast_to(jnp.reshape(v, (1, 1)), (1, _H2))

    contrib = jnp.concatenate([
        jnp.sum(nf, axis=0)[None, :],
        jnp.sum(nf * nf, axis=0)[None, :],
        jnp.sum(lp * nf, axis=0)[None, :],
        jnp.sum((lp * lp) * nf, axis=0)[None, :],
        jnp.sum((lp * lp) * (nf * nf), axis=0)[None, :],
        jnp.sum(ln * u[...], axis=0)[None, :],
        brd(jnp.sum(ln)),
        brd(jnp.sum(ln * ln)),
        brd(jnp.sum(lp * lp)),
        brd(jnp.sum(jnp.where(a0 > 0.5, 1.0, 0.0))),
        jnp.zeros((6, _H2), _f32),
    ], axis=0)

    @pl.when(i == 0)
    def _():
        acc[...] = jnp.zeros((16, _H2), _f32)

    acc[...] += contrib


def _tc2(h1, agg, cnt, u, g, W2l, W2r, b2, Wf1, bf1, Wf2, bf2):
    return pl.pallas_call(
        _tc2_body,
        grid=(_GRID,),
        in_specs=[
            pl.BlockSpec((_BLK, _H1), lambda i: (i, 0)),
            pl.BlockSpec((_NC, _BLK, _H2), lambda i: (0, i, 0)),
            pl.BlockSpec((_NC, _BLK, _D), lambda i: (0, i, 0)),
            pl.BlockSpec((_BLK, _H2), lambda i: (i, 0)),
            pl.BlockSpec((_BLK, 2), lambda i: (i, 0)),
            pl.BlockSpec((_H1, _H2), lambda i: (0, 0)),
            pl.BlockSpec((_H1, _H2), lambda i: (0, 0)),
            pl.BlockSpec((1, _H2), lambda i: (0, 0)),
            pl.BlockSpec((_H2, _F1), lambda i: (0, 0)),
            pl.BlockSpec((1, _F1), lambda i: (0, 0)),
            pl.BlockSpec((_F1, _F2), lambda i: (0, 0)),
            pl.BlockSpec((1, _F2), lambda i: (0, 0)),
        ],
        out_specs=[
            pl.BlockSpec((_BLK, 2), lambda i: (i, 0)),
            pl.BlockSpec((16, _H2), lambda i: (0, 0)),
        ],
        out_shape=[
            jax.ShapeDtypeStruct((_N, 2), _f32),
            jax.ShapeDtypeStruct((16, _H2), _f32),
        ],
    )(h1, agg, cnt, u, g, W2l, W2r, b2, Wf1, bf1, Wf2, bf2)


def kernel(features, edge_index, W1l, W1r, b1, W2l, W2r, b2, Wf1, bf1, Wf2, bf2):
    src = edge_index[0]
    dst = edge_index[1]
    src3 = src.reshape(_NW, _NCHUNK, _CH)
    dst3 = dst.reshape(_NW, _NCHUNK, _CH)
    zf = jnp.zeros((_N, _D), _f32)
    ones_rows = jnp.ones((_CH, _D), _f32)

    cnt = _sc_cnt(dst3, zf, ones_rows)
    agg1 = _sc_agg(features, src3, dst3, zf)
    h1 = _tc1(features, agg1, cnt, W1l, W1r, b1.reshape(1, _H1))
    agg2 = _sc_agg(h1, src3, dst3, zf)

    g = jax.random.gumbel(jax.random.key(1), (_N, 2), dtype=_f32)
    u = jax.random.uniform(jax.random.key(2), (_N, _H2), dtype=_f32)
    assignment, acc = _tc2(h1, agg2, cnt, u, g, W2l, W2r, b2.reshape(1, _H2),
                           Wf1, bf1.reshape(1, _F1), Wf2, bf2.reshape(1, _F2))

    pad = jnp.zeros((_EPAD - _E,), jnp.int32)
    srcp = jnp.concatenate([src, pad]).reshape(_NW, _AIT, 16)
    dstp = jnp.concatenate([dst, pad]).reshape(_NW, _AIT, 16)
    adj_parts = _sc_adj(assignment[:, 0], assignment[:, 1], srcp, dstp)

    s_nf = acc[0]
    s_nf2 = acc[1]
    s_lpnf = acc[2]
    s_lp2nf = acc[3]
    s_lp2nf2 = acc[4]
    s_lnu = acc[5]
    s_ln = acc[6, 0]
    s_ln2 = acc[7, 0]
    s_lp2 = acc[8, 0]
    c05 = acc[9, 0]

    nf_mean = s_nf / _N
    var = jnp.maximum((s_nf2 - _N * nf_mean * nf_mean) / (_N - 1), 0.0)
    nf_std = jnp.sqrt(var)

    graph_feature = s_nf[None, :]
    subgraph_representation = s_lpnf[None, :]
    noisy_graph_feature = (s_lpnf + s_ln * nf_mean + s_lnu * nf_std)[None, :]

    denom = (nf_std + _EPS) ** 2
    q = s_lp2nf2 - 2.0 * nf_mean * s_lp2nf + nf_mean * nf_mean * s_lp2
    kl_loss = jnp.mean((0.5 * s_ln2 * var + _N * q) / denom) / _N

    m = jnp.sum(adj_parts, axis=(0, 2)).reshape(2, 2)
    row_norm = jnp.clip(jnp.sum(jnp.abs(m), axis=1, keepdims=True), 1e-12, None)
    norm_diag = jnp.diag(m / row_norm)
    pos_penalty = jnp.mean((norm_diag - 1.0) ** 2)

    preserve_rate = c05 / _N

    return (graph_feature, noisy_graph_feature, subgraph_representation,
            pos_penalty, kl_loss, preserve_rate)


# speedup vs baseline: 8.7431x; 1.3132x over previous
import jax
import jax.numpy as jnp
from jax import lax
from jax.experimental import pallas as pl
from jax.experimental.pallas import tpu as pltpu
from jax.experimental.pallas import tpu_sc as plsc

_N = 10000
_E = 160000
_D = 128
_H1 = 128
_H2 = 128
_F1 = 64
_F2 = 2
_EPS = 1e-07

_NC = 2
_NS = 16
_NW = _NC * _NS
_EPT = _E // _NW
_CH = 125
_NCHUNK = _EPT // _CH
_RPS = 624
_TBASE = _RPS * _NS
_TAIL = _N - _TBASE
_AIT = 313
_EPAD = _NW * _AIT * 16

_f32 = jnp.float32
_mesh = plsc.VectorSubcoreMesh(core_axis_name="c", subcore_axis_name="s")


def _sc_agg_body(x_hbm, src_hbm, dst_hbm, zf_hbm, out_feat,
                 sidx, didx, rows0, rows1, feat_acc, sem0, sem1):
    cid = lax.axis_index("c")
    sid = lax.axis_index("s")
    wid = sid * _NC + cid
    rb = sid * _RPS
    pltpu.sync_copy(zf_hbm.at[pl.ds(rb, _RPS)], feat_acc.at[pl.ds(rb, _RPS)])

    @pl.when(sid == _NS - 1)
    def _():
        pltpu.sync_copy(zf_hbm.at[pl.ds(_TBASE, _TAIL)],
                        feat_acc.at[pl.ds(_TBASE, _TAIL)])

    pltpu.sync_copy(src_hbm.at[wid], sidx)
    pltpu.sync_copy(dst_hbm.at[wid], didx)
    plsc.subcore_barrier()

    pltpu.async_copy(x_hbm.at[sidx.at[0]], rows0, sem0)

    def pair(p, carry):
        j = 2 * p
        pltpu.async_copy(x_hbm.at[sidx.at[j + 1]], rows1, sem1)
        pltpu.make_async_copy(x_hbm.at[sidx.at[j]], rows0, sem0).wait()
        pltpu.sync_copy(rows0, feat_acc.at[didx.at[j]], add=True)

        @pl.when(p < _NCHUNK // 2 - 1)
        def _():
            pltpu.async_copy(x_hbm.at[sidx.at[j + 2]], rows0, sem0)

        pltpu.make_async_copy(x_hbm.at[sidx.at[j + 1]], rows1, sem1).wait()
        pltpu.sync_copy(rows1, feat_acc.at[didx.at[j + 1]], add=True)
        return carry

    lax.fori_loop(0, _NCHUNK // 2, pair, 0)
    plsc.subcore_barrier()
    pltpu.sync_copy(feat_acc.at[pl.ds(rb, _RPS)],
                    out_feat.at[cid, pl.ds(rb, _RPS)])

    @pl.when(sid == _NS - 1)
    def _():
        pltpu.sync_copy(feat_acc.at[pl.ds(_TBASE, _TAIL)],
                        out_feat.at[cid, pl.ds(_TBASE, _TAIL)])


_sc_agg = pl.kernel(
    _sc_agg_body,
    out_type=jax.ShapeDtypeStruct((_NC, _N, _D), _f32),
    mesh=_mesh,
    scratch_types=[
        pltpu.VMEM((_NCHUNK, _CH), jnp.int32),
        pltpu.VMEM((_NCHUNK, _CH), jnp.int32),
        pltpu.VMEM((_CH, _D), _f32),
        pltpu.VMEM((_CH, _D), _f32),
        pltpu.VMEM_SHARED((_N, _D), _f32),
        pltpu.SemaphoreType.DMA,
        pltpu.SemaphoreType.DMA,
    ],
)


def _sc_cnt_body(dst_hbm, zc_hbm, ones_hbm, out_cnt, didx, ones_b, cnt_acc):
    cid = lax.axis_index("c")
    sid = lax.axis_index("s")
    wid = sid * _NC + cid
    rb = sid * _RPS
    pltpu.sync_copy(zc_hbm.at[pl.ds(rb, _RPS)], cnt_acc.at[pl.ds(rb, _RPS)])

    @pl.when(sid == _NS - 1)
    def _():
        pltpu.sync_copy(zc_hbm.at[pl.ds(_TBASE, _TAIL)],
                        cnt_acc.at[pl.ds(_TBASE, _TAIL)])

    pltpu.sync_copy(ones_hbm, ones_b)
    pltpu.sync_copy(dst_hbm.at[wid], didx)
    plsc.subcore_barrier()

    def chunk(j, carry):
        pltpu.sync_copy(ones_b, cnt_acc.at[didx.at[j]], add=True)
        return carry

    lax.fori_loop(0, _NCHUNK, chunk, 0)
    plsc.subcore_barrier()
    pltpu.sync_copy(cnt_acc.at[pl.ds(rb, _RPS)],
                    out_cnt.at[cid, pl.ds(rb, _RPS)])

    @pl.when(sid == _NS - 1)
    def _():
        pltpu.sync_copy(cnt_acc.at[pl.ds(_TBASE, _TAIL)],
                        out_cnt.at[cid, pl.ds(_TBASE, _TAIL)])


_sc_cnt = pl.kernel(
    _sc_cnt_body,
    out_type=jax.ShapeDtypeStruct((_NC, _N, _D), _f32),
    mesh=_mesh,
    scratch_types=[
        pltpu.VMEM((_NCHUNK, _CH), jnp.int32),
        pltpu.VMEM((_CH, _D), _f32),
        pltpu.VMEM_SHARED((_N, _D), _f32),
    ],
)


def _sc_adj_body(a0_hbm, a1_hbm, src_hbm, dst_hbm, out_hbm,
                 a0b, a1b, sbuf, dbuf, outb):
    cid = lax.axis_index("c")
    sid = lax.axis_index("s")
    wid = sid * _NC + cid
    pltpu.sync_copy(a0_hbm, a0b)
    pltpu.sync_copy(a1_hbm, a1b)
    pltpu.sync_copy(src_hbm.at[wid], sbuf)
    pltpu.sync_copy(dst_hbm.at[wid], dbuf)
    lane = lax.iota(jnp.int32, 16)
    ebase = wid * (_AIT * 16)

    def it(i, carry):
        s00, s01, s10, s11 = carry
        idx_s = sbuf[i, :]
        idx_d = dbuf[i, :]
        a0s = plsc.load_gather(a0b, [idx_s])
        a1s = plsc.load_gather(a1b, [idx_s])
        a0d = plsc.load_gather(a0b, [idx_d])
        a1d = plsc.load_gather(a1b, [idx_d])
        w = jnp.where(ebase + i * 16 + lane < _E, 1.0, 0.0).astype(_f32)
        a0dw = a0d * w
        a1dw = a1d * w
        return (s00 + a0s * a0dw, s01 + a0s * a1dw,
                s10 + a1s * a0dw, s11 + a1s * a1dw)

    z = jnp.zeros((16,), _f32)
    s00, s01, s10, s11 = lax.fori_loop(0, _AIT, it, (z, z, z, z))
    outb[0, :] = s00
    outb[1, :] = s01
    outb[2, :] = s10
    outb[3, :] = s11
    pltpu.sync_copy(outb, out_hbm.at[wid])


_sc_adj = pl.kernel(
    _sc_adj_body,
    out_type=jax.ShapeDtypeStruct((_NW, 4, 16), _f32),
    mesh=_mesh,
    compiler_params=pltpu.CompilerParams(needs_layout_passes=False),
    scratch_types=[
        pltpu.VMEM((_N,), _f32),
        pltpu.VMEM((_N,), _f32),
        pltpu.VMEM((_AIT, 16), jnp.int32),
        pltpu.VMEM((_AIT, 16), jnp.int32),
        pltpu.VMEM((4, 16), _f32),
    ],
)

_BLK = 1000
_GRID = _N // _BLK


def _tc1_body(x, agg, cnt, W1l, W1r, b1, out):
    c = jnp.maximum(cnt[0, :, 0:1] + cnt[1, :, 0:1], 1.0)
    mean = (agg[0] + agg[1]) / c
    h = (jnp.dot(mean, W1l[...], preferred_element_type=_f32)
         + jnp.dot(x[...], W1r[...], preferred_element_type=_f32) + b1[...])
    out[...] = jnp.maximum(h, 0.0)


def _tc1(x, agg, cnt, W1l, W1r, b1):
    return pl.pallas_call(
        _tc1_body,
        grid=(_GRID,),
        in_specs=[
            pl.BlockSpec((_BLK, _D), lambda i: (i, 0)),
            pl.BlockSpec((_NC, _BLK, _D), lambda i: (0, i, 0)),
            pl.BlockSpec((_NC, _BLK, _D), lambda i: (0, i, 0)),
            pl.BlockSpec((_D, _H1), lambda i: (0, 0)),
            pl.BlockSpec((_D, _H1), lambda i: (0, 0)),
            pl.BlockSpec((1, _H1), lambda i: (0, 0)),
        ],
        out_specs=pl.BlockSpec((_BLK, _H1), lambda i: (i, 0)),
        out_shape=jax.ShapeDtypeStruct((_N, _H1), _f32),
    )(x, agg, cnt, W1l, W1r, b1)


def _tc2_body(h1, agg, cnt, u, g, W2l, W2r, b2, Wf1, bf1, Wf2, bf2,
              assign, acc):
    i = pl.program_id(0)
    c = jnp.maximum(cnt[0, :, 0:1] + cnt[1, :, 0:1], 1.0)
    mean = (agg[0] + agg[1]) / c
    nf = (jnp.dot(mean, W2l[...], preferred_element_type=_f32)
          + jnp.dot(h1[...], W2r[...], preferred_element_type=_f32) + b2[...])
    ab = jnp.tanh(jnp.dot(nf, Wf1[...], preferred_element_type=_f32) + bf1[...])
    lg = jnp.dot(ab, Wf2[...], preferred_element_type=_f32) + bf2[...]
    l0 = lg[:, 0:1]
    l1 = lg[:, 1:2]
    m = jnp.maximum(l0, l1)
    e0 = jnp.exp(l0 - m)
    e1 = jnp.exp(l1 - m)
    s = e0 + e1
    a0 = e0 / s
    a1 = e1 / s
    assign[...] = jnp.concatenate([a0, a1], axis=1)
    q0 = a0 + g[:, 0:1]
    q1 = a1 + g[:, 1:2]
    mq = jnp.maximum(q0, q1)
    f0 = jnp.exp(q0 - mq)
    f1 = jnp.exp(q1 - mq)
    fs = f0 + f1
    lp = f0 / fs
    ln = f1 / fs

    def brd(v):
        return jnp.broadcast_to(jnp.reshape(v, (1, 1)), (1, _H2))

    contrib = jnp.concatenate([
        jnp.sum(nf, axis=0)[None, :],
        jnp.sum(nf * nf, axis=0)[None, :],
        jnp.sum(lp * nf, axis=0)[None, :],
        jnp.sum((lp * lp) * nf, axis=0)[None, :],
        jnp.sum((lp * lp) * (nf * nf), axis=0)[None, :],
        jnp.sum(ln * u[...], axis=0)[None, :],
        brd(jnp.sum(ln)),
        brd(jnp.sum(ln * ln)),
        brd(jnp.sum(lp * lp)),
        brd(jnp.sum(jnp.where(a0 > 0.5, 1.0, 0.0))),
        jnp.zeros((6, _H2), _f32),
    ], axis=0)

    @pl.when(i == 0)
    def _():
        acc[...] = jnp.zeros((16, _H2), _f32)

    acc[...] += contrib


def _tc2(h1, agg, cnt, u, g, W2l, W2r, b2, Wf1, bf1, Wf2, bf2):
    return pl.pallas_call(
        _tc2_body,
        grid=(_GRID,),
        in_specs=[
            pl.BlockSpec((_BLK, _H1), lambda i: (i, 0)),
            pl.BlockSpec((_NC, _BLK, _H2), lambda i: (0, i, 0)),
            pl.BlockSpec((_NC, _BLK, _D), lambda i: (0, i, 0)),
            pl.BlockSpec((_BLK, _H2), lambda i: (i, 0)),
            pl.BlockSpec((_BLK, 2), lambda i: (i, 0)),
            pl.BlockSpec((_H1, _H2), lambda i: (0, 0)),
            pl.BlockSpec((_H1, _H2), lambda i: (0, 0)),
            pl.BlockSpec((1, _H2), lambda i: (0, 0)),
            pl.BlockSpec((_H2, _F1), lambda i: (0, 0)),
            pl.BlockSpec((1, _F1), lambda i: (0, 0)),
            pl.BlockSpec((_F1, _F2), lambda i: (0, 0)),
            pl.BlockSpec((1, _F2), lambda i: (0, 0)),
        ],
        out_specs=[
            pl.BlockSpec((_BLK, 2), lambda i: (i, 0)),
            pl.BlockSpec((16, _H2), lambda i: (0, 0)),
        ],
        out_shape=[
            jax.ShapeDtypeStruct((_N, 2), _f32),
            jax.ShapeDtypeStruct((16, _H2), _f32),
        ],
    )(h1, agg, cnt, u, g, W2l, W2r, b2, Wf1, bf1, Wf2, bf2)


def kernel(features, edge_index, W1l, W1r, b1, W2l, W2r, b2, Wf1, bf1, Wf2, bf2):
    src = edge_index[0]
    dst = edge_index[1]
    src3 = src.reshape(_NW, _NCHUNK, _CH)
    dst3 = dst.reshape(_NW, _NCHUNK, _CH)
    zf = jnp.zeros((_N, _D), _f32)
    ones_rows = jnp.ones((_CH, _D), _f32)

    cnt = _sc_cnt(dst3, zf, ones_rows)
    agg1 = _sc_agg(features, src3, dst3, zf)
    h1 = _tc1(features, agg1, cnt, W1l, W1r, b1.reshape(1, _H1))
    agg2 = _sc_agg(h1, src3, dst3, zf)

    g = jax.random.gumbel(jax.random.key(1), (_N, 2), dtype=_f32)
    u = jax.random.uniform(jax.random.key(2), (_N, _H2), dtype=_f32)
    assignment, acc = _tc2(h1, agg2, cnt, u, g, W2l, W2r, b2.reshape(1, _H2),
                           Wf1, bf1.reshape(1, _F1), Wf2, bf2.reshape(1, _F2))

    pad = jnp.zeros((_EPAD - _E,), jnp.int32)
    srcp = jnp.concatenate([src, pad]).reshape(_NW, _AIT, 16)
    dstp = jnp.concatenate([dst, pad]).reshape(_NW, _AIT, 16)
    adj_parts = _sc_adj(assignment[:, 0], assignment[:, 1], srcp, dstp)

    s_nf = acc[0]
    s_nf2 = acc[1]
    s_lpnf = acc[2]
    s_lp2nf = acc[3]
    s_lp2nf2 = acc[4]
    s_lnu = acc[5]
    s_ln = acc[6, 0]
    s_ln2 = acc[7, 0]
    s_lp2 = acc[8, 0]
    c05 = acc[9, 0]

    nf_mean = s_nf / _N
    var = jnp.maximum((s_nf2 - _N * nf_mean * nf_mean) / (_N - 1), 0.0)
    nf_std = jnp.sqrt(var)

    graph_feature = s_nf[None, :]
    subgraph_representation = s_lpnf[None, :]
    noisy_graph_feature = (s_lpnf + s_ln * nf_mean + s_lnu * nf_std)[None, :]

    denom = (nf_std + _EPS) ** 2
    q = s_lp2nf2 - 2.0 * nf_mean * s_lp2nf + nf_mean * nf_mean * s_lp2
    kl_loss = jnp.mean((0.5 * s_ln2 * var + _N * q) / denom) / _N

    m = jnp.sum(adj_parts, axis=(0, 2)).reshape(2, 2)
    row_norm = jnp.clip(jnp.sum(jnp.abs(m), axis=1, keepdims=True), 1e-12, None)
    norm_diag = jnp.diag(m / row_norm)
    pos_penalty = jnp.mean((norm_diag - 1.0) ** 2)

    preserve_rate = c05 / _N

    return (graph_feature, noisy_graph_feature, subgraph_representation,
            pos_penalty, kl_loss, preserve_rate)


# speedup vs baseline: 10.2980x; 1.1778x over previous
import jax
import jax.numpy as jnp
from jax import lax
from jax.experimental import pallas as pl
from jax.experimental.pallas import tpu as pltpu
from jax.experimental.pallas import tpu_sc as plsc

_N = 10000
_E = 160000
_D = 128
_H1 = 128
_H2 = 128
_F1 = 64
_F2 = 2
_EPS = 1e-07

_NC = 2
_NS = 16
_NW = _NC * _NS
_EPT = _E // _NW
_CH = 125
_NCHUNK = _EPT // _CH
_RPS = 624
_TBASE = _RPS * _NS
_TAIL = _N - _TBASE
_AIT = 320
_EPAD = _NW * _AIT * 16

_f32 = jnp.float32
_mesh = plsc.VectorSubcoreMesh(core_axis_name="c", subcore_axis_name="s")


def _make_sc_agg(with_cnt: bool):
    scratch = [
        pltpu.VMEM((_NCHUNK, _CH), jnp.int32),
        pltpu.VMEM((_NCHUNK, _CH), jnp.int32),
        pltpu.VMEM((_CH, _D), _f32),
        pltpu.VMEM((_CH, _D), _f32),
        pltpu.VMEM_SHARED((_N, _D), _f32),
        pltpu.SemaphoreType.DMA,
        pltpu.SemaphoreType.DMA,
    ]
    del with_cnt
    out_type = [jax.ShapeDtypeStruct((_NC, _N, _D), _f32)]

    def body(x_hbm, src_hbm, dst_hbm, zf_hbm, *rest):
        out_feat, sidx, didx, rows0, rows1, feat_acc, sem0, sem1 = rest
        cid = lax.axis_index("c")
        sid = lax.axis_index("s")
        wid = sid * _NC + cid
        rb = sid * _RPS
        pltpu.sync_copy(zf_hbm.at[pl.ds(rb, _RPS)],
                        feat_acc.at[pl.ds(rb, _RPS)])

        @pl.when(sid == _NS - 1)
        def _():
            pltpu.sync_copy(zf_hbm.at[pl.ds(_TBASE, _TAIL)],
                            feat_acc.at[pl.ds(_TBASE, _TAIL)])

        pltpu.sync_copy(src_hbm.at[wid], sidx)
        pltpu.sync_copy(dst_hbm.at[wid], didx)
        plsc.subcore_barrier()

        pltpu.async_copy(x_hbm.at[sidx.at[0]], rows0, sem0)

        def pair(p, carry):
            j = 2 * p
            pltpu.async_copy(x_hbm.at[sidx.at[j + 1]], rows1, sem1)
            pltpu.make_async_copy(x_hbm.at[sidx.at[j]], rows0, sem0).wait()
            pltpu.sync_copy(rows0, feat_acc.at[didx.at[j]], add=True)

            @pl.when(p < _NCHUNK // 2 - 1)
            def _():
                pltpu.async_copy(x_hbm.at[sidx.at[j + 2]], rows0, sem0)

            pltpu.make_async_copy(x_hbm.at[sidx.at[j + 1]], rows1, sem1).wait()
            pltpu.sync_copy(rows1, feat_acc.at[didx.at[j + 1]], add=True)
            return carry

        lax.fori_loop(0, _NCHUNK // 2, pair, 0)
        plsc.subcore_barrier()
        pltpu.sync_copy(feat_acc.at[pl.ds(rb, _RPS)],
                        out_feat.at[cid, pl.ds(rb, _RPS)])

        @pl.when(sid == _NS - 1)
        def _():
            pltpu.sync_copy(feat_acc.at[pl.ds(_TBASE, _TAIL)],
                            out_feat.at[cid, pl.ds(_TBASE, _TAIL)])

    return pl.kernel(
        body,
        out_type=out_type[0],
        mesh=_mesh,
        scratch_types=scratch,
    )


_sc_agg = _make_sc_agg(False)


def _sc_cnt_body(dstp_hbm, z80_hbm, out_cnt, dpad, cntbuf):
    cid = lax.axis_index("c")
    sid = lax.axis_index("s")
    wid = sid * _NC + cid
    pltpu.sync_copy(dstp_hbm.at[wid], dpad)
    pltpu.sync_copy(z80_hbm, cntbuf)
    lane = lax.iota(jnp.int32, 16)
    ones16 = jnp.ones((16,), _f32)
    ebase = wid * (_AIT * 16)

    def cstep(i, carry):
        idx = dpad[i >> 3, pl.ds((i & 7) * 16, 16)]
        msk = ebase + i * 16 + lane < _E
        plsc.addupdate_scatter(cntbuf, [idx >> 7, idx & 127], ones16,
                               mask=msk)
        return carry

    lax.fori_loop(0, _AIT, cstep, 0)
    pltpu.sync_copy(cntbuf, out_cnt.at[wid])


_sc_cnt = pl.kernel(
    _sc_cnt_body,
    out_type=jax.ShapeDtypeStruct((_NW, 80, _D), _f32),
    mesh=_mesh,
    scratch_types=[
        pltpu.VMEM((_AIT // 8, _D), jnp.int32),
        pltpu.VMEM((80, _D), _f32),
    ],
    compiler_params=pltpu.CompilerParams(needs_layout_passes=False),
)


def _sc_adj_body(a0_hbm, a1_hbm, src_hbm, dst_hbm, out_hbm,
                 a0b, a1b, sbuf, dbuf, outb):
    cid = lax.axis_index("c")
    sid = lax.axis_index("s")
    wid = sid * _NC + cid
    pltpu.sync_copy(a0_hbm, a0b)
    pltpu.sync_copy(a1_hbm, a1b)
    pltpu.sync_copy(src_hbm.at[wid], sbuf)
    pltpu.sync_copy(dst_hbm.at[wid], dbuf)
    lane = lax.iota(jnp.int32, 16)
    ebase = wid * (_AIT * 16)

    def it(i, carry):
        s00, s01, s10, s11 = carry
        idx_s = sbuf[i >> 3, pl.ds((i & 7) * 16, 16)]
        idx_d = dbuf[i >> 3, pl.ds((i & 7) * 16, 16)]
        a0s = plsc.load_gather(a0b, [idx_s])
        a1s = plsc.load_gather(a1b, [idx_s])
        a0d = plsc.load_gather(a0b, [idx_d])
        a1d = plsc.load_gather(a1b, [idx_d])
        w = jnp.where(ebase + i * 16 + lane < _E, 1.0, 0.0).astype(_f32)
        a0dw = a0d * w
        a1dw = a1d * w
        return (s00 + a0s * a0dw, s01 + a0s * a1dw,
                s10 + a1s * a0dw, s11 + a1s * a1dw)

    z = jnp.zeros((16,), _f32)
    s00, s01, s10, s11 = lax.fori_loop(0, _AIT, it, (z, z, z, z))
    outb[0, :] = s00
    outb[1, :] = s01
    outb[2, :] = s10
    outb[3, :] = s11
    pltpu.sync_copy(outb, out_hbm.at[wid])


_sc_adj = pl.kernel(
    _sc_adj_body,
    out_type=jax.ShapeDtypeStruct((_NW, 4, 16), _f32),
    mesh=_mesh,
    compiler_params=pltpu.CompilerParams(needs_layout_passes=False),
    scratch_types=[
        pltpu.VMEM((_N,), _f32),
        pltpu.VMEM((_N,), _f32),
        pltpu.VMEM((_AIT // 8, _D), jnp.int32),
        pltpu.VMEM((_AIT // 8, _D), jnp.int32),
        pltpu.VMEM((4, 16), _f32),
    ],
)

_BLK = 1280
_GRID = 8
_GR = _BLK // _D


def _count_col(c80, i):
    sel = ((lax.broadcasted_iota(jnp.int32, (_BLK, 80), 0) >> 7) + _GR * i
           == lax.broadcasted_iota(jnp.int32, (_BLK, 80), 1)).astype(_f32)
    tmp = jnp.dot(sel, c80, preferred_element_type=_f32)
    lmask = ((lax.broadcasted_iota(jnp.int32, (_BLK, _D), 0) & 127)
             == lax.broadcasted_iota(jnp.int32, (_BLK, _D), 1)).astype(_f32)
    return jnp.sum(tmp * lmask, axis=1, keepdims=True)


def _tc1_body(x, agg, cnt, W1l, W1r, b1, out):
    c = jnp.maximum(_count_col(jnp.sum(cnt[...], axis=0), pl.program_id(0)),
                    1.0)
    mean = (agg[0] + agg[1]) / c
    h = (jnp.dot(mean, W1l[...], preferred_element_type=_f32)
         + jnp.dot(x[...], W1r[...], preferred_element_type=_f32) + b1[...])
    out[...] = jnp.maximum(h, 0.0)


def _tc1(x, agg, cnt, W1l, W1r, b1):
    return pl.pallas_call(
        _tc1_body,
        grid=(_GRID,),
        in_specs=[
            pl.BlockSpec((_BLK, _D), lambda i: (i, 0)),
            pl.BlockSpec((_NC, _BLK, _D), lambda i: (0, i, 0)),
            pl.BlockSpec((_NW, 80, _D), lambda i: (0, 0, 0)),
            pl.BlockSpec((_D, _H1), lambda i: (0, 0)),
            pl.BlockSpec((_D, _H1), lambda i: (0, 0)),
            pl.BlockSpec((1, _H1), lambda i: (0, 0)),
        ],
        out_specs=pl.BlockSpec((_BLK, _H1), lambda i: (i, 0)),
        out_shape=jax.ShapeDtypeStruct((_N, _H1), _f32),
    )(x, agg, cnt, W1l, W1r, b1)


def _tc2_body(h1, agg, cnt, u, g, W2l, W2r, b2, Wf1, bf1, Wf2, bf2,
              assign, acc):
    i = pl.program_id(0)
    c = jnp.maximum(_count_col(jnp.sum(cnt[...], axis=0), i), 1.0)
    mean = (agg[0] + agg[1]) / c
    nf = (jnp.dot(mean, W2l[...], preferred_element_type=_f32)
          + jnp.dot(h1[...], W2r[...], preferred_element_type=_f32) + b2[...])
    ab = jnp.tanh(jnp.dot(nf, Wf1[...], preferred_element_type=_f32) + bf1[...])
    lg = jnp.dot(ab, Wf2[...], preferred_element_type=_f32) + bf2[...]
    l0 = lg[:, 0:1]
    l1 = lg[:, 1:2]
    m = jnp.maximum(l0, l1)
    e0 = jnp.exp(l0 - m)
    e1 = jnp.exp(l1 - m)
    s = e0 + e1
    a0 = e0 / s
    a1 = e1 / s
    assign[...] = jnp.concatenate([a0, a1], axis=1)
    q0 = a0 + g[:, 0:1]
    q1 = a1 + g[:, 1:2]
    mq = jnp.maximum(q0, q1)
    f0 = jnp.exp(q0 - mq)
    f1 = jnp.exp(q1 - mq)
    fs = f0 + f1
    lp = f0 / fs
    ln = f1 / fs

    vm = (i * _BLK
          + lax.broadcasted_iota(jnp.int32, (_BLK, 1), 0)) < _N
    nfm = jnp.where(vm, nf, 0.0)
    lpm = jnp.where(vm, lp, 0.0)
    lnm = jnp.where(vm, ln, 0.0)
    lnu = jnp.where(vm, ln * u[...], 0.0)

    def brd(v):
        return jnp.broadcast_to(jnp.reshape(v, (1, 1)), (1, _H2))

    contrib = jnp.concatenate([
        jnp.sum(nfm, axis=0)[None, :],
        jnp.sum(nfm * nfm, axis=0)[None, :],
        jnp.sum(lpm * nfm, axis=0)[None, :],
        jnp.sum((lpm * lpm) * nfm, axis=0)[None, :],
        jnp.sum((lpm * lpm) * (nfm * nfm), axis=0)[None, :],
        jnp.sum(lnu, axis=0)[None, :],
        brd(jnp.sum(lnm)),
        brd(jnp.sum(lnm * lnm)),
        brd(jnp.sum(lpm * lpm)),
        brd(jnp.sum(jnp.where(vm & (a0 > 0.5), 1.0, 0.0))),
        jnp.zeros((6, _H2), _f32),
    ], axis=0)

    @pl.when(i == 0)
    def _():
        acc[...] = jnp.zeros((16, _H2), _f32)

    acc[...] += contrib


def _tc2(h1, agg, cnt, u, g, W2l, W2r, b2, Wf1, bf1, Wf2, bf2):
    return pl.pallas_call(
        _tc2_body,
        grid=(_GRID,),
        in_specs=[
            pl.BlockSpec((_BLK, _H1), lambda i: (i, 0)),
            pl.BlockSpec((_NC, _BLK, _H2), lambda i: (0, i, 0)),
            pl.BlockSpec((_NW, 80, _D), lambda i: (0, 0, 0)),
            pl.BlockSpec((_BLK, _H2), lambda i: (i, 0)),
            pl.BlockSpec((_BLK, 2), lambda i: (i, 0)),
            pl.BlockSpec((_H1, _H2), lambda i: (0, 0)),
            pl.BlockSpec((_H1, _H2), lambda i: (0, 0)),
            pl.BlockSpec((1, _H2), lambda i: (0, 0)),
            pl.BlockSpec((_H2, _F1), lambda i: (0, 0)),
            pl.BlockSpec((1, _F1), lambda i: (0, 0)),
            pl.BlockSpec((_F1, _F2), lambda i: (0, 0)),
            pl.BlockSpec((1, _F2), lambda i: (0, 0)),
        ],
        out_specs=[
            pl.BlockSpec((_BLK, 2), lambda i: (i, 0)),
            pl.BlockSpec((16, _H2), lambda i: (0, 0)),
        ],
        out_shape=[
            jax.ShapeDtypeStruct((_N, 2), _f32),
            jax.ShapeDtypeStruct((16, _H2), _f32),
        ],
    )(h1, agg, cnt, u, g, W2l, W2r, b2, Wf1, bf1, Wf2, bf2)


def kernel(features, edge_index, W1l, W1r, b1, W2l, W2r, b2, Wf1, bf1, Wf2, bf2):
    src = edge_index[0]
    dst = edge_index[1]
    src3 = src.reshape(_NW, _NCHUNK, _CH)
    dst3 = dst.reshape(_NW, _NCHUNK, _CH)
    zf = jnp.zeros((_N, _D), _f32)
    z80 = jnp.zeros((80, _D), _f32)
    pad = jnp.zeros((_EPAD - _E,), jnp.int32)
    srcp = jnp.concatenate([src, pad]).reshape(_NW, _AIT // 8, _D)
    dstp = jnp.concatenate([dst, pad]).reshape(_NW, _AIT // 8, _D)

    cnt = _sc_cnt(dstp, z80)
    agg1 = _sc_agg(features, src3, dst3, zf)
    h1 = _tc1(features, agg1, cnt, W1l, W1r, b1.reshape(1, _H1))
    agg2 = _sc_agg(h1, src3, dst3, zf)

    g = jax.random.gumbel(jax.random.key(1), (_N, 2), dtype=_f32)
    u = jax.random.uniform(jax.random.key(2), (_N, _H2), dtype=_f32)
    assignment, acc = _tc2(h1, agg2, cnt, u, g, W2l, W2r, b2.reshape(1, _H2),
                           Wf1, bf1.reshape(1, _F1), Wf2, bf2.reshape(1, _F2))

    adj_parts = _sc_adj(assignment[:, 0], assignment[:, 1], srcp, dstp)

    s_nf = acc[0]
    s_nf2 = acc[1]
    s_lpnf = acc[2]
    s_lp2nf = acc[3]
    s_lp2nf2 = acc[4]
    s_lnu = acc[5]
    s_ln = acc[6, 0]
    s_ln2 = acc[7, 0]
    s_lp2 = acc[8, 0]
    c05 = acc[9, 0]

    nf_mean = s_nf / _N
    var = jnp.maximum((s_nf2 - _N * nf_mean * nf_mean) / (_N - 1), 0.0)
    nf_std = jnp.sqrt(var)

    graph_feature = s_nf[None, :]
    subgraph_representation = s_lpnf[None, :]
    noisy_graph_feature = (s_lpnf + s_ln * nf_mean + s_lnu * nf_std)[None, :]

    denom = (nf_std + _EPS) ** 2
    q = s_lp2nf2 - 2.0 * nf_mean * s_lp2nf + nf_mean * nf_mean * s_lp2
    kl_loss = jnp.mean((0.5 * s_ln2 * var + _N * q) / denom) / _N

    m = jnp.sum(adj_parts, axis=(0, 2)).reshape(2, 2)
    row_norm = jnp.clip(jnp.sum(jnp.abs(m), axis=1, keepdims=True), 1e-12, None)
    norm_diag = jnp.diag(m / row_norm)
    pos_penalty = jnp.mean((norm_diag - 1.0) ** 2)

    preserve_rate = c05 / _N

    return (graph_feature, noisy_graph_feature, subgraph_representation,
            pos_penalty, kl_loss, preserve_rate)
